# baseline (speedup 1.0000x reference)
import functools

import jax
import jax.numpy as jnp
import numpy as np
from jax import lax
from jax.experimental import pallas as pl
from jax.experimental.pallas import tpu as pltpu
from jax.experimental.pallas import tpu_sc as plsc

N = 10000
E = 160000
D = 256
DH = 128
G = 64

NT = 16
EPT = E // NT
K = 80
NCH = EPT // K


ZR = 624
ZREM = N - ZR * NT

_PERM128 = np.arange(DH).reshape(DH // 32, 2, 16).transpose(0, 2, 1).reshape(DH)
_EPERM = np.concatenate([_PERM128, _PERM128 + DH])
NPAIR = NCH // 2


def _edge_phase(hA, hB, eA, eB, src, dst, zrows):
    mesh = plsc.VectorSubcoreMesh(core_axis_name="c", subcore_axis_name="s")

    @functools.partial(
        pl.kernel,
        mesh=mesh,
        compiler_params=pltpu.CompilerParams(needs_layout_passes=False),
        out_type=(
            jax.ShapeDtypeStruct((N, DH), jnp.float32),
            jax.ShapeDtypeStruct((N, DH), jnp.float32),
        ),
        scratch_types=[
            pltpu.VMEM((K,), jnp.int32),
            pltpu.VMEM((K,), jnp.int32),
            pltpu.VMEM((K,), jnp.int32),
            pltpu.VMEM((K,), jnp.int32),
            pltpu.VMEM((K, DH), jnp.float32),
            pltpu.VMEM((K, DH), jnp.float32),
            pltpu.VMEM((K * DH // 2,), jnp.int32),
            pltpu.VMEM((K * DH // 2,), jnp.int32),
            pltpu.VMEM_SHARED((N, DH), jnp.float32),
            pltpu.SemaphoreType.DMA,
            pltpu.SemaphoreType.DMA,
            pltpu.SemaphoreType.DMA,
            pltpu.SemaphoreType.DMA,
            pltpu.SemaphoreType.DMA,
            pltpu.SemaphoreType.DMA,
        ],
    )
    def edge_k(hA_h, hB_h, eA_h, eB_h, src_h, dst_h, z_h, aA_h, aB_h,
               siA, diA, siB, diB, gA, gB, ebA, ebB, accum,
               gsA, esA, scA, gsB, esB, scB):
        c = lax.axis_index("c")
        s = lax.axis_index("s")

        def run(h_h, e_h, a_h):
            rb = pl.multiple_of(s * ZR, 8)
            pltpu.sync_copy(z_h.at[pl.ds(rb, ZR)], accum.at[pl.ds(rb, ZR)])

            @pl.when(s == NT - 1)
            def _():
                pltpu.sync_copy(z_h.at[pl.ds(ZR * NT, ZREM)],
                                accum.at[pl.ds(ZR * NT, ZREM)])

            plsc.subcore_barrier()

            base0 = s * EPT

            def load_idx(k, si, di):
                b = pl.multiple_of(base0 + k * K, 16)
                pltpu.sync_copy(src_h.at[pl.ds(b, K)], si)
                pltpu.sync_copy(dst_h.at[pl.ds(b, K)], di)

            def issue(k, si, gb, eb, gsem, esem):
                b = pl.multiple_of((base0 + k * K) * (DH // 2), 1024)
                pltpu.async_copy(h_h.at[si], gb, gsem)
                pltpu.async_copy(e_h.at[pl.ds(b, K * DH // 2)], eb, esem)

            def compute(gb, eb):
                hmask = jnp.full((16,), -65536, jnp.int32)

                @plsc.parallel_loop(0, K, 1, unroll=4)
                def _(r):
                    for j in range(DH // 32):
                        pk = eb[pl.ds(r * (DH // 2) + 16 * j, 16)]
                        ea = plsc.bitcast(pk << 16, jnp.float32)
                        eo = plsc.bitcast(pk & hmask, jnp.float32)
                        s0 = pl.ds(32 * j, 16)
                        s1 = pl.ds(32 * j + 16, 16)
                        gb[r, s0] = jnp.maximum(gb[r, s0] + ea, 0.0)
                        gb[r, s1] = jnp.maximum(gb[r, s1] + eo, 0.0)

            load_idx(0, siA, diA)
            issue(0, siA, gA, ebA, gsA, esA)

            def pair(j, carry):
                k0 = 2 * j

                @pl.when(j > 0)
                def _():
                    pltpu.make_async_copy(gB, accum.at[diB], scB).wait()

                load_idx(k0 + 1, siB, diB)
                issue(k0 + 1, siB, gB, ebB, gsB, esB)

                pltpu.make_async_copy(h_h.at[siA], gA, gsA).wait()
                pltpu.make_async_copy(e_h.at[pl.ds(0, K * DH // 2)], ebA, esA).wait()
                compute(gA, ebA)
                pltpu.async_copy(gA, accum.at[diA], scA, add=True)

                pltpu.make_async_copy(h_h.at[siB], gB, gsB).wait()
                pltpu.make_async_copy(e_h.at[pl.ds(0, K * DH // 2)], ebB, esB).wait()
                compute(gB, ebB)
                pltpu.async_copy(gB, accum.at[diB], scB, add=True)

                pltpu.make_async_copy(gA, accum.at[diA], scA).wait()
                load_idx(k0 + 2, siA, diA)
                issue(k0 + 2, siA, gA, ebA, gsA, esA)
                return carry

            lax.fori_loop(0, NPAIR, pair, 0)

            pltpu.make_async_copy(gB, accum.at[diB], scB).wait()
            pltpu.make_async_copy(h_h.at[siA], gA, gsA).wait()
            pltpu.make_async_copy(e_h.at[pl.ds(0, K * DH // 2)], ebA, esA).wait()
            compute(gA, ebA)
            pltpu.async_copy(gA, accum.at[diA], scA, add=True)
            pltpu.make_async_copy(gA, accum.at[diA], scA).wait()

            plsc.subcore_barrier()
            pltpu.sync_copy(accum.at[pl.ds(rb, ZR)], a_h.at[pl.ds(rb, ZR)])

            @pl.when(s == NT - 1)
            def _():
                pltpu.sync_copy(accum.at[pl.ds(ZR * NT, ZREM)],
                                a_h.at[pl.ds(ZR * NT, ZREM)])

        @pl.when(c == 0)
        def _():
            run(hA_h, eA_h, aA_h)

        @pl.when(c == 1)
        def _():
            run(hB_h, eB_h, aB_h)

    return edge_k(hA, hB, eA, eB, src, dst, zrows)


def _enc_body(x_ref, wA_ref, wB_ref, b_ref, hA_ref, hB_ref):
    x = x_ref[...]
    b = b_ref[...]
    hA_ref[...] = jnp.dot(x, wA_ref[...], preferred_element_type=jnp.float32) + b[:, :DH]
    hB_ref[...] = jnp.dot(x, wB_ref[...], preferred_element_type=jnp.float32) + b[:, DH:]


def _node_encode(x, We, be):
    R = 2000
    return pl.pallas_call(
        _enc_body,
        grid=(N // R,),
        in_specs=[
            pl.BlockSpec((R, D), lambda i: (i, 0)),
            pl.BlockSpec((D, DH), lambda i: (0, 0)),
            pl.BlockSpec((D, DH), lambda i: (0, 0)),
            pl.BlockSpec((1, D), lambda i: (0, 0)),
        ],
        out_specs=[
            pl.BlockSpec((R, DH), lambda i: (i, 0)),
            pl.BlockSpec((R, DH), lambda i: (i, 0)),
        ],
        out_shape=[
            jax.ShapeDtypeStruct((N, DH), jnp.float32),
            jax.ShapeDtypeStruct((N, DH), jnp.float32),
        ],
    )(x, We[:, :DH], We[:, DH:], be.reshape(1, D))


def _edge_enc_body(x_ref, wA_ref, wB_ref, b_ref, eA_ref, eB_ref):
    x = x_ref[...]
    b = b_ref[...]
    eA_ref[...] = (jnp.dot(x, wA_ref[...], preferred_element_type=jnp.float32)
                   + b[:, :DH]).astype(jnp.bfloat16)
    eB_ref[...] = (jnp.dot(x, wB_ref[...], preferred_element_type=jnp.float32)
                   + b[:, DH:]).astype(jnp.bfloat16)


def _edge_encode(edge_attr, Wee, bee):
    R = 4000
    de = Wee.shape[0]
    return pl.pallas_call(
        _edge_enc_body,
        grid=(E // R,),
        in_specs=[
            pl.BlockSpec((R, de), lambda i: (i, 0)),
            pl.BlockSpec((de, DH), lambda i: (0, 0)),
            pl.BlockSpec((de, DH), lambda i: (0, 0)),
            pl.BlockSpec((1, D), lambda i: (0, 0)),
        ],
        out_specs=[
            pl.BlockSpec((R, DH), lambda i: (i, 0)),
            pl.BlockSpec((R, DH), lambda i: (i, 0)),
        ],
        out_shape=[
            jax.ShapeDtypeStruct((E, DH), jnp.bfloat16),
            jax.ShapeDtypeStruct((E, DH), jnp.bfloat16),
        ],
    )(edge_attr, Wee[:, :DH], Wee[:, DH:], bee.reshape(1, D))


def _k1_body(hA_ref, hB_ref, aA_ref, aB_ref, wcA_ref, wcB_ref, bc_ref,
             w1_ref, b1_ref, z_ref, s_ref, q_ref):
    uA = hA_ref[...] + aA_ref[...]
    uB = hB_ref[...] + aB_ref[...]
    hc = (jnp.dot(uA, wcA_ref[...], preferred_element_type=jnp.float32)
          + jnp.dot(uB, wcB_ref[...], preferred_element_type=jnp.float32)
          + bc_ref[...])
    z = jnp.dot(hc, w1_ref[...], preferred_element_type=jnp.float32) + b1_ref[...]
    z_ref[...] = z

    @pl.when(pl.program_id(0) == 0)
    def _():
        s_ref[...] = jnp.zeros_like(s_ref)
        q_ref[...] = jnp.zeros_like(q_ref)

    s_ref[...] += jnp.sum(z, axis=0, keepdims=True)
    q_ref[...] += jnp.sum(z * z, axis=0, keepdims=True)


def _layer_k1(hA, hB, aA, aB, Wc, bc, W1, b1):
    R = 2000
    D2 = 2 * D
    return pl.pallas_call(
        _k1_body,
        grid=(N // R,),
        in_specs=[
            pl.BlockSpec((R, DH), lambda i: (i, 0)),
            pl.BlockSpec((R, DH), lambda i: (i, 0)),
            pl.BlockSpec((R, DH), lambda i: (i, 0)),
            pl.BlockSpec((R, DH), lambda i: (i, 0)),
            pl.BlockSpec((DH, D), lambda i: (0, 0)),
            pl.BlockSpec((DH, D), lambda i: (0, 0)),
            pl.BlockSpec((1, D), lambda i: (0, 0)),
            pl.BlockSpec((D, D2), lambda i: (0, 0)),
            pl.BlockSpec((1, D2), lambda i: (0, 0)),
        ],
        out_specs=[
            pl.BlockSpec((R, D2), lambda i: (i, 0)),
            pl.BlockSpec((1, D2), lambda i: (0, 0)),
            pl.BlockSpec((1, D2), lambda i: (0, 0)),
        ],
        out_shape=[
            jax.ShapeDtypeStruct((N, D2), jnp.float32),
            jax.ShapeDtypeStruct((1, D2), jnp.float32),
            jax.ShapeDtypeStruct((1, D2), jnp.float32),
        ],
    )(hA, hB, aA, aB, Wc[:DH, :], Wc[DH:, :], bc.reshape(1, D),
      W1, b1.reshape(1, D2))


def _k2_body(z_ref, s_ref, q_ref, gam_ref, bet_ref, w2A_ref, w2B_ref,
             b2_ref, hA_ref, hB_ref, oA_ref, oB_ref):
    mu = s_ref[...] / N
    var = q_ref[...] / N - mu * mu
    inv = lax.rsqrt(var + 1e-5)
    zn = (z_ref[...] - mu) * (inv * gam_ref[...]) + bet_ref[...]
    zl = jnp.where(zn >= 0, zn, 0.01 * zn)
    b2 = b2_ref[...]
    oA_ref[...] = (jnp.dot(zl, w2A_ref[...], preferred_element_type=jnp.float32)
                   + b2[:, :DH] + hA_ref[...])
    oB_ref[...] = (jnp.dot(zl, w2B_ref[...], preferred_element_type=jnp.float32)
                   + b2[:, DH:] + hB_ref[...])


def _layer_k2(z, ssum, ssq, gam, bet, W2, b2, hA, hB):
    R = 2000
    D2 = 2 * D
    return pl.pallas_call(
        _k2_body,
        grid=(N // R,),
        in_specs=[
            pl.BlockSpec((R, D2), lambda i: (i, 0)),
            pl.BlockSpec((1, D2), lambda i: (0, 0)),
            pl.BlockSpec((1, D2), lambda i: (0, 0)),
            pl.BlockSpec((1, D2), lambda i: (0, 0)),
            pl.BlockSpec((1, D2), lambda i: (0, 0)),
            pl.BlockSpec((D2, DH), lambda i: (0, 0)),
            pl.BlockSpec((D2, DH), lambda i: (0, 0)),
            pl.BlockSpec((1, D), lambda i: (0, 0)),
            pl.BlockSpec((R, DH), lambda i: (i, 0)),
            pl.BlockSpec((R, DH), lambda i: (i, 0)),
        ],
        out_specs=[
            pl.BlockSpec((R, DH), lambda i: (i, 0)),
            pl.BlockSpec((R, DH), lambda i: (i, 0)),
        ],
        out_shape=[
            jax.ShapeDtypeStruct((N, DH), jnp.float32),
            jax.ShapeDtypeStruct((N, DH), jnp.float32),
        ],
    )(z, ssum, ssq, gam.reshape(1, D2), bet.reshape(1, D2),
      W2[:, :DH], W2[:, DH:], b2.reshape(1, D), hA, hB)


def _pool_body(hA_ref, hB_ref, b_ref, wgA_ref, wgB_ref, bg_ref,
               wfA_ref, wfB_ref, bf_ref, o_ref):
    hA = hA_ref[...]
    hB = hB_ref[...]
    gate = (jnp.sum(hA * wgA_ref[...], axis=1, keepdims=True)
            + jnp.sum(hB * wgB_ref[...], axis=1, keepdims=True)
            + bg_ref[0, 0])
    seg = b_ref[...]
    gids = lax.broadcasted_iota(jnp.int32, (N, G), 1)
    M = seg == gids
    gm = jnp.max(jnp.where(M, gate, jnp.float32(-1e30)), axis=0, keepdims=True)
    gmax_n = jnp.sum(jnp.where(M, gm, 0.0), axis=1, keepdims=True)
    ex = jnp.exp(gate - gmax_n)
    den = jnp.sum(jnp.where(M, ex, 0.0), axis=0, keepdims=True)
    den_n = jnp.sum(jnp.where(M, den, 0.0), axis=1, keepdims=True)
    w = ex / den_n
    Mf = M.astype(jnp.float32)
    dn = (((0,), (0,)), ((), ()))
    pA = lax.dot_general(Mf, hA * w, dn, preferred_element_type=jnp.float32)
    pB = lax.dot_general(Mf, hB * w, dn, preferred_element_type=jnp.float32)
    t = (jnp.dot(pA, wfA_ref[...], preferred_element_type=jnp.float32)
         + jnp.dot(pB, wfB_ref[...], preferred_element_type=jnp.float32)
         + bf_ref[0, 0])
    o_ref[...] = 1.0 / (1.0 + jnp.exp(-t))


def _pool(hA, hB, batch32, Wg, bg, Wf, bf):
    return pl.pallas_call(
        _pool_body,
        out_shape=jax.ShapeDtypeStruct((G, 1), jnp.float32),
    )(hA, hB, batch32, Wg[:DH, 0].reshape(1, DH), Wg[DH:, 0].reshape(1, DH),
      bg.reshape(1, 1),
      Wf[:DH, :], Wf[DH:, :], bf.reshape(1, 1))


def kernel(x, edge_index, edge_attr, batch, We, be, Wee, bee, Wc, bc,
           W1, b1, gam, bet, W2, b2, Wg, bg, Wf, bf):
    src = edge_index[0].astype(jnp.int32)
    dst = edge_index[1].astype(jnp.int32)
    batch32 = batch.astype(jnp.int32).reshape(N, 1)
    zrows = jnp.zeros((N, DH), jnp.float32)

    hA, hB = _node_encode(x, We, be)
    eA, eB = _edge_encode(edge_attr, Wee[:, _EPERM], bee[_EPERM])
    eA = lax.bitcast_convert_type(eA.reshape(E * DH // 2, 2), jnp.int32)
    eB = lax.bitcast_convert_type(eB.reshape(E * DH // 2, 2), jnp.int32)

    for i in range(Wc.shape[0]):
        aA, aB = _edge_phase(hA, hB, eA, eB, src, dst, zrows)
        z, ssum, ssq = _layer_k1(hA, hB, aA, aB, Wc[i], bc[i], W1[i], b1[i])
        hA, hB = _layer_k2(z, ssum, ssq, gam[i], bet[i], W2[i], b2[i], hA, hB)

    return _pool(hA, hB, batch32, Wg, bg, Wf, bf)


# speedup vs baseline: 11.2381x; 11.2381x over previous
import functools

import jax
import jax.numpy as jnp
import numpy as np
from jax import lax
from jax.experimental import pallas as pl
from jax.experimental.pallas import tpu as pltpu
from jax.experimental.pallas import tpu_sc as plsc

N = 10000
E = 160000
D = 256
DH = 128
G = 64

NT = 16
EPT = E // NT
K = 80
NCH = EPT // K


ZR = 624
ZREM = N - ZR * NT

_PERM128 = np.arange(DH).reshape(DH // 32, 2, 16).transpose(0, 2, 1).reshape(DH)
_LO = _PERM128[0::2]
_HI = _PERM128[1::2]
NPAIR = NCH // 2


def _edge_phase(hA, hB, eA, eB, src, dst, zrows):
    mesh = plsc.VectorSubcoreMesh(core_axis_name="c", subcore_axis_name="s")

    @functools.partial(
        pl.kernel,
        mesh=mesh,
        compiler_params=pltpu.CompilerParams(needs_layout_passes=False),
        out_type=(
            jax.ShapeDtypeStruct((N, DH), jnp.float32),
            jax.ShapeDtypeStruct((N, DH), jnp.float32),
        ),
        scratch_types=[
            pltpu.VMEM((K,), jnp.int32),
            pltpu.VMEM((K,), jnp.int32),
            pltpu.VMEM((K,), jnp.int32),
            pltpu.VMEM((K,), jnp.int32),
            pltpu.VMEM((K, DH), jnp.float32),
            pltpu.VMEM((K, DH), jnp.float32),
            pltpu.VMEM((K, DH // 2), jnp.int32),
            pltpu.VMEM((K, DH // 2), jnp.int32),
            pltpu.VMEM_SHARED((N, DH), jnp.float32),
            pltpu.SemaphoreType.DMA,
            pltpu.SemaphoreType.DMA,
            pltpu.SemaphoreType.DMA,
            pltpu.SemaphoreType.DMA,
            pltpu.SemaphoreType.DMA,
            pltpu.SemaphoreType.DMA,
        ],
    )
    def edge_k(hA_h, hB_h, eA_h, eB_h, src_h, dst_h, z_h, aA_h, aB_h,
               siA, diA, siB, diB, gA, gB, ebA, ebB, accum,
               gsA, esA, scA, gsB, esB, scB):
        c = lax.axis_index("c")
        s = lax.axis_index("s")

        def run(h_h, e_h, a_h):
            rb = pl.multiple_of(s * ZR, 8)
            pltpu.sync_copy(z_h.at[pl.ds(rb, ZR)], accum.at[pl.ds(rb, ZR)])

            @pl.when(s == NT - 1)
            def _():
                pltpu.sync_copy(z_h.at[pl.ds(ZR * NT, ZREM)],
                                accum.at[pl.ds(ZR * NT, ZREM)])

            plsc.subcore_barrier()

            base0 = s * EPT

            def load_idx(k, si, di):
                b = pl.multiple_of(base0 + k * K, 16)
                pltpu.sync_copy(src_h.at[pl.ds(b, K)], si)
                pltpu.sync_copy(dst_h.at[pl.ds(b, K)], di)

            def issue(k, si, gb, eb, gsem, esem):
                b = pl.multiple_of(base0 + k * K, 16)
                pltpu.async_copy(h_h.at[si], gb, gsem)
                pltpu.async_copy(e_h.at[pl.ds(b, K)], eb, esem)

            def compute(gb, eb):
                hmask = jnp.full((16,), -65536, jnp.int32)

                @plsc.parallel_loop(0, K, 1, unroll=4)
                def _(r):
                    for j in range(DH // 32):
                        pk = eb[r, pl.ds(16 * j, 16)]
                        ea = plsc.bitcast(pk << 16, jnp.float32)
                        eo = plsc.bitcast(pk & hmask, jnp.float32)
                        s0 = pl.ds(32 * j, 16)
                        s1 = pl.ds(32 * j + 16, 16)
                        gb[r, s0] = jnp.maximum(gb[r, s0] + ea, 0.0)
                        gb[r, s1] = jnp.maximum(gb[r, s1] + eo, 0.0)

            load_idx(0, siA, diA)
            issue(0, siA, gA, ebA, gsA, esA)

            def pair(j, carry):
                k0 = 2 * j

                @pl.when(j > 0)
                def _():
                    pltpu.make_async_copy(gB, accum.at[diB], scB).wait()

                load_idx(k0 + 1, siB, diB)
                issue(k0 + 1, siB, gB, ebB, gsB, esB)

                pltpu.make_async_copy(h_h.at[siA], gA, gsA).wait()
                pltpu.make_async_copy(e_h.at[pl.ds(0, K)], ebA, esA).wait()
                compute(gA, ebA)
                pltpu.async_copy(gA, accum.at[diA], scA, add=True)

                pltpu.make_async_copy(h_h.at[siB], gB, gsB).wait()
                pltpu.make_async_copy(e_h.at[pl.ds(0, K)], ebB, esB).wait()
                compute(gB, ebB)
                pltpu.async_copy(gB, accum.at[diB], scB, add=True)

                pltpu.make_async_copy(gA, accum.at[diA], scA).wait()
                load_idx(k0 + 2, siA, diA)
                issue(k0 + 2, siA, gA, ebA, gsA, esA)
                return carry

            lax.fori_loop(0, NPAIR, pair, 0)

            pltpu.make_async_copy(gB, accum.at[diB], scB).wait()
            pltpu.make_async_copy(h_h.at[siA], gA, gsA).wait()
            pltpu.make_async_copy(e_h.at[pl.ds(0, K)], ebA, esA).wait()
            compute(gA, ebA)
            pltpu.async_copy(gA, accum.at[diA], scA, add=True)
            pltpu.make_async_copy(gA, accum.at[diA], scA).wait()

            plsc.subcore_barrier()
            pltpu.sync_copy(accum.at[pl.ds(rb, ZR)], a_h.at[pl.ds(rb, ZR)])

            @pl.when(s == NT - 1)
            def _():
                pltpu.sync_copy(accum.at[pl.ds(ZR * NT, ZREM)],
                                a_h.at[pl.ds(ZR * NT, ZREM)])

        @pl.when(c == 0)
        def _():
            run(hA_h, eA_h, aA_h)

        @pl.when(c == 1)
        def _():
            run(hB_h, eB_h, aB_h)

    return edge_k(hA, hB, eA, eB, src, dst, zrows)


def _enc_body(x_ref, wA_ref, wB_ref, b_ref, hA_ref, hB_ref):
    x = x_ref[...]
    b = b_ref[...]
    hA_ref[...] = jnp.dot(x, wA_ref[...], preferred_element_type=jnp.float32) + b[:, :DH]
    hB_ref[...] = jnp.dot(x, wB_ref[...], preferred_element_type=jnp.float32) + b[:, DH:]


def _node_encode(x, We, be):
    R = 2000
    return pl.pallas_call(
        _enc_body,
        grid=(N // R,),
        in_specs=[
            pl.BlockSpec((R, D), lambda i: (i, 0)),
            pl.BlockSpec((D, DH), lambda i: (0, 0)),
            pl.BlockSpec((D, DH), lambda i: (0, 0)),
            pl.BlockSpec((1, D), lambda i: (0, 0)),
        ],
        out_specs=[
            pl.BlockSpec((R, DH), lambda i: (i, 0)),
            pl.BlockSpec((R, DH), lambda i: (i, 0)),
        ],
        out_shape=[
            jax.ShapeDtypeStruct((N, DH), jnp.float32),
            jax.ShapeDtypeStruct((N, DH), jnp.float32),
        ],
    )(x, We[:, :DH], We[:, DH:], be.reshape(1, D))


def _edge_enc_body(x_ref, wAl_ref, wAh_ref, wBl_ref, wBh_ref, b_ref,
                   eA_ref, eB_ref):
    x = x_ref[...]
    b = b_ref[...]

    def pack(wl, wh, bl, bh):
        lo = jnp.dot(x, wl, preferred_element_type=jnp.float32) + bl
        hi = jnp.dot(x, wh, preferred_element_type=jnp.float32) + bh
        lo16 = lax.bitcast_convert_type(lo.astype(jnp.bfloat16),
                                        jnp.uint16).astype(jnp.int32)
        hi16 = lax.bitcast_convert_type(hi.astype(jnp.bfloat16),
                                        jnp.uint16).astype(jnp.int32)
        return lo16 | (hi16 << 16)

    q = DH // 2
    eA_ref[...] = pack(wAl_ref[...], wAh_ref[...], b[:, :q], b[:, q:2 * q])
    eB_ref[...] = pack(wBl_ref[...], wBh_ref[...], b[:, 2 * q:3 * q], b[:, 3 * q:])


def _edge_encode(edge_attr, Wee, bee):
    R = 4000
    de = Wee.shape[0]
    q = DH // 2
    Wl = [Wee[:, _LO], Wee[:, _HI], Wee[:, _LO + DH], Wee[:, _HI + DH]]
    bq = jnp.concatenate([bee[_LO], bee[_HI], bee[_LO + DH], bee[_HI + DH]])
    return pl.pallas_call(
        _edge_enc_body,
        grid=(E // R,),
        in_specs=[
            pl.BlockSpec((R, de), lambda i: (i, 0)),
            pl.BlockSpec((de, q), lambda i: (0, 0)),
            pl.BlockSpec((de, q), lambda i: (0, 0)),
            pl.BlockSpec((de, q), lambda i: (0, 0)),
            pl.BlockSpec((de, q), lambda i: (0, 0)),
            pl.BlockSpec((1, D), lambda i: (0, 0)),
        ],
        out_specs=[
            pl.BlockSpec((R, q), lambda i: (i, 0)),
            pl.BlockSpec((R, q), lambda i: (i, 0)),
        ],
        out_shape=[
            jax.ShapeDtypeStruct((E, q), jnp.int32),
            jax.ShapeDtypeStruct((E, q), jnp.int32),
        ],
    )(edge_attr, Wl[0], Wl[1], Wl[2], Wl[3], bq.reshape(1, D))


def _k1_body(hA_ref, hB_ref, aA_ref, aB_ref, wcA_ref, wcB_ref, bc_ref,
             w1_ref, b1_ref, z_ref, s_ref, q_ref):
    uA = hA_ref[...] + aA_ref[...]
    uB = hB_ref[...] + aB_ref[...]
    hc = (jnp.dot(uA, wcA_ref[...], preferred_element_type=jnp.float32)
          + jnp.dot(uB, wcB_ref[...], preferred_element_type=jnp.float32)
          + bc_ref[...])
    z = jnp.dot(hc, w1_ref[...], preferred_element_type=jnp.float32) + b1_ref[...]
    z_ref[...] = z

    @pl.when(pl.program_id(0) == 0)
    def _():
        s_ref[...] = jnp.zeros_like(s_ref)
        q_ref[...] = jnp.zeros_like(q_ref)

    s_ref[...] += jnp.sum(z, axis=0, keepdims=True)
    q_ref[...] += jnp.sum(z * z, axis=0, keepdims=True)


def _layer_k1(hA, hB, aA, aB, Wc, bc, W1, b1):
    R = 2000
    D2 = 2 * D
    return pl.pallas_call(
        _k1_body,
        grid=(N // R,),
        in_specs=[
            pl.BlockSpec((R, DH), lambda i: (i, 0)),
            pl.BlockSpec((R, DH), lambda i: (i, 0)),
            pl.BlockSpec((R, DH), lambda i: (i, 0)),
            pl.BlockSpec((R, DH), lambda i: (i, 0)),
            pl.BlockSpec((DH, D), lambda i: (0, 0)),
            pl.BlockSpec((DH, D), lambda i: (0, 0)),
            pl.BlockSpec((1, D), lambda i: (0, 0)),
            pl.BlockSpec((D, D2), lambda i: (0, 0)),
            pl.BlockSpec((1, D2), lambda i: (0, 0)),
        ],
        out_specs=[
            pl.BlockSpec((R, D2), lambda i: (i, 0)),
            pl.BlockSpec((1, D2), lambda i: (0, 0)),
            pl.BlockSpec((1, D2), lambda i: (0, 0)),
        ],
        out_shape=[
            jax.ShapeDtypeStruct((N, D2), jnp.float32),
            jax.ShapeDtypeStruct((1, D2), jnp.float32),
            jax.ShapeDtypeStruct((1, D2), jnp.float32),
        ],
    )(hA, hB, aA, aB, Wc[:DH, :], Wc[DH:, :], bc.reshape(1, D),
      W1, b1.reshape(1, D2))


def _k2_body(z_ref, s_ref, q_ref, gam_ref, bet_ref, w2A_ref, w2B_ref,
             b2_ref, hA_ref, hB_ref, oA_ref, oB_ref):
    mu = s_ref[...] / N
    var = q_ref[...] / N - mu * mu
    inv = lax.rsqrt(var + 1e-5)
    zn = (z_ref[...] - mu) * (inv * gam_ref[...]) + bet_ref[...]
    zl = jnp.where(zn >= 0, zn, 0.01 * zn)
    b2 = b2_ref[...]
    oA_ref[...] = (jnp.dot(zl, w2A_ref[...], preferred_element_type=jnp.float32)
                   + b2[:, :DH] + hA_ref[...])
    oB_ref[...] = (jnp.dot(zl, w2B_ref[...], preferred_element_type=jnp.float32)
                   + b2[:, DH:] + hB_ref[...])


def _layer_k2(z, ssum, ssq, gam, bet, W2, b2, hA, hB):
    R = 2000
    D2 = 2 * D
    return pl.pallas_call(
        _k2_body,
        grid=(N // R,),
        in_specs=[
            pl.BlockSpec((R, D2), lambda i: (i, 0)),
            pl.BlockSpec((1, D2), lambda i: (0, 0)),
            pl.BlockSpec((1, D2), lambda i: (0, 0)),
            pl.BlockSpec((1, D2), lambda i: (0, 0)),
            pl.BlockSpec((1, D2), lambda i: (0, 0)),
            pl.BlockSpec((D2, DH), lambda i: (0, 0)),
            pl.BlockSpec((D2, DH), lambda i: (0, 0)),
            pl.BlockSpec((1, D), lambda i: (0, 0)),
            pl.BlockSpec((R, DH), lambda i: (i, 0)),
            pl.BlockSpec((R, DH), lambda i: (i, 0)),
        ],
        out_specs=[
            pl.BlockSpec((R, DH), lambda i: (i, 0)),
            pl.BlockSpec((R, DH), lambda i: (i, 0)),
        ],
        out_shape=[
            jax.ShapeDtypeStruct((N, DH), jnp.float32),
            jax.ShapeDtypeStruct((N, DH), jnp.float32),
        ],
    )(z, ssum, ssq, gam.reshape(1, D2), bet.reshape(1, D2),
      W2[:, :DH], W2[:, DH:], b2.reshape(1, D), hA, hB)


def _pool_body(hA_ref, hB_ref, b_ref, wgA_ref, wgB_ref, bg_ref,
               wfA_ref, wfB_ref, bf_ref, o_ref):
    hA = hA_ref[...]
    hB = hB_ref[...]
    gate = (jnp.sum(hA * wgA_ref[...], axis=1, keepdims=True)
            + jnp.sum(hB * wgB_ref[...], axis=1, keepdims=True)
            + bg_ref[0, 0])
    seg = b_ref[...]
    gids = lax.broadcasted_iota(jnp.int32, (N, G), 1)
    M = seg == gids
    gm = jnp.max(jnp.where(M, gate, jnp.float32(-1e30)), axis=0, keepdims=True)
    gmax_n = jnp.sum(jnp.where(M, gm, 0.0), axis=1, keepdims=True)
    ex = jnp.exp(gate - gmax_n)
    den = jnp.sum(jnp.where(M, ex, 0.0), axis=0, keepdims=True)
    den_n = jnp.sum(jnp.where(M, den, 0.0), axis=1, keepdims=True)
    w = ex / den_n
    Mf = M.astype(jnp.float32)
    dn = (((0,), (0,)), ((), ()))
    pA = lax.dot_general(Mf, hA * w, dn, preferred_element_type=jnp.float32)
    pB = lax.dot_general(Mf, hB * w, dn, preferred_element_type=jnp.float32)
    t = (jnp.dot(pA, wfA_ref[...], preferred_element_type=jnp.float32)
         + jnp.dot(pB, wfB_ref[...], preferred_element_type=jnp.float32)
         + bf_ref[0, 0])
    o_ref[...] = 1.0 / (1.0 + jnp.exp(-t))


def _pool(hA, hB, batch32, Wg, bg, Wf, bf):
    return pl.pallas_call(
        _pool_body,
        out_shape=jax.ShapeDtypeStruct((G, 1), jnp.float32),
    )(hA, hB, batch32, Wg[:DH, 0].reshape(1, DH), Wg[DH:, 0].reshape(1, DH),
      bg.reshape(1, 1),
      Wf[:DH, :], Wf[DH:, :], bf.reshape(1, 1))


def kernel(x, edge_index, edge_attr, batch, We, be, Wee, bee, Wc, bc,
           W1, b1, gam, bet, W2, b2, Wg, bg, Wf, bf):
    src = edge_index[0].astype(jnp.int32)
    dst = edge_index[1].astype(jnp.int32)
    batch32 = batch.astype(jnp.int32).reshape(N, 1)
    zrows = jnp.zeros((N, DH), jnp.float32)

    hA, hB = _node_encode(x, We, be)
    eA, eB = _edge_encode(edge_attr, Wee, bee)

    for i in range(Wc.shape[0]):
        aA, aB = _edge_phase(hA, hB, eA, eB, src, dst, zrows)
        z, ssum, ssq = _layer_k1(hA, hB, aA, aB, Wc[i], bc[i], W1[i], b1[i])
        hA, hB = _layer_k2(z, ssum, ssq, gam[i], bet[i], W2[i], b2[i], hA, hB)

    return _pool(hA, hB, batch32, Wg, bg, Wf, bf)


# speedup vs baseline: 13.1036x; 1.1660x over previous
import functools

import jax
import jax.numpy as jnp
import numpy as np
from jax import lax
from jax.experimental import pallas as pl
from jax.experimental.pallas import tpu as pltpu
from jax.experimental.pallas import tpu_sc as plsc

N = 10000
E = 160000
D = 256
DH = 128
G = 64

NT = 16
EPT = E // NT
K = 80
NCH = EPT // K


ZR = 624
ZREM = N - ZR * NT

_PERM128 = np.arange(DH).reshape(DH // 32, 2, 16).transpose(0, 2, 1).reshape(DH)
_LO = _PERM128[0::2]
_HI = _PERM128[1::2]
SUP = 25


def _edge_phase(hA, hB, eA, eB, sdx, zrows):
    mesh = plsc.VectorSubcoreMesh(core_axis_name="c", subcore_axis_name="s")

    @functools.partial(
        pl.kernel,
        mesh=mesh,
        compiler_params=pltpu.CompilerParams(needs_layout_passes=False),
        out_type=(
            jax.ShapeDtypeStruct((N, DH), jnp.float32),
            jax.ShapeDtypeStruct((N, DH), jnp.float32),
        ),
        scratch_types=[
            pltpu.VMEM((SUP, 2, K), jnp.int32),
            pltpu.VMEM((K, DH), jnp.float32),
            pltpu.VMEM((K, DH), jnp.float32),
            pltpu.VMEM((K, DH // 2), jnp.int32),
            pltpu.VMEM((K, DH // 2), jnp.int32),
            pltpu.VMEM_SHARED((N, DH), jnp.float32),
            pltpu.SemaphoreType.DMA,
            pltpu.SemaphoreType.DMA,
            pltpu.SemaphoreType.DMA,
            pltpu.SemaphoreType.DMA,
            pltpu.SemaphoreType.DMA,
            pltpu.SemaphoreType.DMA,
        ],
    )
    def edge_k(hA_h, hB_h, eA_h, eB_h, sdx_h, z_h, aA_h, aB_h,
               sdv, gA, gB, ebA, ebB, accum,
               gsA, esA, scA, gsB, esB, scB):
        c = lax.axis_index("c")
        s = lax.axis_index("s")

        def run(h_h, e_h, a_h):
            rb = pl.multiple_of(s * ZR, 8)
            pltpu.sync_copy(z_h.at[pl.ds(rb, ZR)], accum.at[pl.ds(rb, ZR)])

            @pl.when(s == NT - 1)
            def _():
                pltpu.sync_copy(z_h.at[pl.ds(ZR * NT, ZREM)],
                                accum.at[pl.ds(ZR * NT, ZREM)])

            plsc.subcore_barrier()

            base0 = s * EPT

            def issue(kg, t, gb, eb, gsem, esem):
                b = pl.multiple_of(base0 + kg * K, 16)
                pltpu.async_copy(h_h.at[sdv.at[t, 0]], gb, gsem)
                pltpu.async_copy(e_h.at[pl.ds(b, K)], eb, esem)

            def scatter(t, gb, ssem):
                pltpu.async_copy(gb, accum.at[sdv.at[t, 1]], ssem, add=True)

            def compute(gb, eb):
                hmask = jnp.full((16,), -65536, jnp.int32)

                @plsc.parallel_loop(0, K, 1, unroll=4)
                def _(r):
                    for j in range(DH // 32):
                        pk = eb[r, pl.ds(16 * j, 16)]
                        ea = plsc.bitcast(pk << 16, jnp.float32)
                        eo = plsc.bitcast(pk & hmask, jnp.float32)
                        s0 = pl.ds(32 * j, 16)
                        s1 = pl.ds(32 * j + 16, 16)
                        gb[r, s0] = jnp.maximum(gb[r, s0] + ea, 0.0)
                        gb[r, s1] = jnp.maximum(gb[r, s1] + eo, 0.0)

            def wait_in(gb, eb, gsem, esem):
                pltpu.make_async_copy(h_h.at[sdv.at[0, 0]], gb, gsem).wait()
                pltpu.make_async_copy(e_h.at[pl.ds(0, K)], eb, esem).wait()

            def wait_sc(t, gb, ssem):
                pltpu.make_async_copy(gb, accum.at[sdv.at[t, 1]], ssem).wait()

            def super_body(s5, carry):
                pltpu.sync_copy(sdx_h.at[pl.ds(s * NCH + s5 * SUP, SUP)], sdv)
                cb = s5 * SUP
                issue(cb, 0, gA, ebA, gsA, esA)

                def pair(j, cc):
                    t0 = 2 * j

                    @pl.when(j > 0)
                    def _():
                        wait_sc(t0 - 1, gB, scB)

                    issue(cb + t0 + 1, t0 + 1, gB, ebB, gsB, esB)

                    wait_in(gA, ebA, gsA, esA)
                    compute(gA, ebA)
                    scatter(t0, gA, scA)

                    wait_in(gB, ebB, gsB, esB)
                    compute(gB, ebB)
                    scatter(t0 + 1, gB, scB)

                    wait_sc(t0, gA, scA)
                    issue(cb + t0 + 2, t0 + 2, gA, ebA, gsA, esA)
                    return cc

                lax.fori_loop(0, SUP // 2, pair, 0)

                wait_sc(SUP - 2, gB, scB)
                wait_in(gA, ebA, gsA, esA)
                compute(gA, ebA)
                scatter(SUP - 1, gA, scA)
                wait_sc(SUP - 1, gA, scA)
                return carry

            lax.fori_loop(0, NCH // SUP, super_body, 0)

            plsc.subcore_barrier()
            pltpu.sync_copy(accum.at[pl.ds(rb, ZR)], a_h.at[pl.ds(rb, ZR)])

            @pl.when(s == NT - 1)
            def _():
                pltpu.sync_copy(accum.at[pl.ds(ZR * NT, ZREM)],
                                a_h.at[pl.ds(ZR * NT, ZREM)])

        @pl.when(c == 0)
        def _():
            run(hA_h, eA_h, aA_h)

        @pl.when(c == 1)
        def _():
            run(hB_h, eB_h, aB_h)

    return edge_k(hA, hB, eA, eB, sdx, zrows)


def _enc_body(x_ref, wA_ref, wB_ref, b_ref, hA_ref, hB_ref):
    x = x_ref[...]
    b = b_ref[...]
    hA_ref[...] = jnp.dot(x, wA_ref[...], preferred_element_type=jnp.float32) + b[:, :DH]
    hB_ref[...] = jnp.dot(x, wB_ref[...], preferred_element_type=jnp.float32) + b[:, DH:]


def _node_encode(x, We, be):
    R = 2000
    return pl.pallas_call(
        _enc_body,
        grid=(N // R,),
        in_specs=[
            pl.BlockSpec((R, D), lambda i: (i, 0)),
            pl.BlockSpec((D, DH), lambda i: (0, 0)),
            pl.BlockSpec((D, DH), lambda i: (0, 0)),
            pl.BlockSpec((1, D), lambda i: (0, 0)),
        ],
        out_specs=[
            pl.BlockSpec((R, DH), lambda i: (i, 0)),
            pl.BlockSpec((R, DH), lambda i: (i, 0)),
        ],
        out_shape=[
            jax.ShapeDtypeStruct((N, DH), jnp.float32),
            jax.ShapeDtypeStruct((N, DH), jnp.float32),
        ],
    )(x, We[:, :DH], We[:, DH:], be.reshape(1, D))


def _edge_enc_body(x_ref, wAl_ref, wAh_ref, wBl_ref, wBh_ref, b_ref,
                   eA_ref, eB_ref):
    x = x_ref[...]
    b = b_ref[...]

    def pack(wl, wh, bl, bh):
        lo = jnp.dot(x, wl, preferred_element_type=jnp.float32) + bl
        hi = jnp.dot(x, wh, preferred_element_type=jnp.float32) + bh
        lo16 = lax.bitcast_convert_type(lo.astype(jnp.bfloat16),
                                        jnp.uint16).astype(jnp.int32)
        hi16 = lax.bitcast_convert_type(hi.astype(jnp.bfloat16),
                                        jnp.uint16).astype(jnp.int32)
        return lo16 | (hi16 << 16)

    q = DH // 2
    eA_ref[...] = pack(wAl_ref[...], wAh_ref[...], b[:, :q], b[:, q:2 * q])
    eB_ref[...] = pack(wBl_ref[...], wBh_ref[...], b[:, 2 * q:3 * q], b[:, 3 * q:])


def _edge_encode(edge_attr, Wee, bee):
    R = 4000
    de = Wee.shape[0]
    q = DH // 2
    Wl = [Wee[:, _LO], Wee[:, _HI], Wee[:, _LO + DH], Wee[:, _HI + DH]]
    bq = jnp.concatenate([bee[_LO], bee[_HI], bee[_LO + DH], bee[_HI + DH]])
    return pl.pallas_call(
        _edge_enc_body,
        grid=(E // R,),
        in_specs=[
            pl.BlockSpec((R, de), lambda i: (i, 0)),
            pl.BlockSpec((de, q), lambda i: (0, 0)),
            pl.BlockSpec((de, q), lambda i: (0, 0)),
            pl.BlockSpec((de, q), lambda i: (0, 0)),
            pl.BlockSpec((de, q), lambda i: (0, 0)),
            pl.BlockSpec((1, D), lambda i: (0, 0)),
        ],
        out_specs=[
            pl.BlockSpec((R, q), lambda i: (i, 0)),
            pl.BlockSpec((R, q), lambda i: (i, 0)),
        ],
        out_shape=[
            jax.ShapeDtypeStruct((E, q), jnp.int32),
            jax.ShapeDtypeStruct((E, q), jnp.int32),
        ],
    )(edge_attr, Wl[0], Wl[1], Wl[2], Wl[3], bq.reshape(1, D))


def _k1_body(hA_ref, hB_ref, aA_ref, aB_ref, wcA_ref, wcB_ref, bc_ref,
             w1_ref, b1_ref, z_ref, s_ref, q_ref):
    uA = hA_ref[...] + aA_ref[...]
    uB = hB_ref[...] + aB_ref[...]
    hc = (jnp.dot(uA, wcA_ref[...], preferred_element_type=jnp.float32)
          + jnp.dot(uB, wcB_ref[...], preferred_element_type=jnp.float32)
          + bc_ref[...])
    z = jnp.dot(hc, w1_ref[...], preferred_element_type=jnp.float32) + b1_ref[...]
    z_ref[...] = z

    @pl.when(pl.program_id(0) == 0)
    def _():
        s_ref[...] = jnp.zeros_like(s_ref)
        q_ref[...] = jnp.zeros_like(q_ref)

    s_ref[...] += jnp.sum(z, axis=0, keepdims=True)
    q_ref[...] += jnp.sum(z * z, axis=0, keepdims=True)


def _layer_k1(hA, hB, aA, aB, Wc, bc, W1, b1):
    R = 2000
    D2 = 2 * D
    return pl.pallas_call(
        _k1_body,
        grid=(N // R,),
        in_specs=[
            pl.BlockSpec((R, DH), lambda i: (i, 0)),
            pl.BlockSpec((R, DH), lambda i: (i, 0)),
            pl.BlockSpec((R, DH), lambda i: (i, 0)),
            pl.BlockSpec((R, DH), lambda i: (i, 0)),
            pl.BlockSpec((DH, D), lambda i: (0, 0)),
            pl.BlockSpec((DH, D), lambda i: (0, 0)),
            pl.BlockSpec((1, D), lambda i: (0, 0)),
            pl.BlockSpec((D, D2), lambda i: (0, 0)),
            pl.BlockSpec((1, D2), lambda i: (0, 0)),
        ],
        out_specs=[
            pl.BlockSpec((R, D2), lambda i: (i, 0)),
            pl.BlockSpec((1, D2), lambda i: (0, 0)),
            pl.BlockSpec((1, D2), lambda i: (0, 0)),
        ],
        out_shape=[
            jax.ShapeDtypeStruct((N, D2), jnp.float32),
            jax.ShapeDtypeStruct((1, D2), jnp.float32),
            jax.ShapeDtypeStruct((1, D2), jnp.float32),
        ],
    )(hA, hB, aA, aB, Wc[:DH, :], Wc[DH:, :], bc.reshape(1, D),
      W1, b1.reshape(1, D2))


def _k2_body(z_ref, s_ref, q_ref, gam_ref, bet_ref, w2A_ref, w2B_ref,
             b2_ref, hA_ref, hB_ref, oA_ref, oB_ref):
    mu = s_ref[...] / N
    var = q_ref[...] / N - mu * mu
    inv = lax.rsqrt(var + 1e-5)
    zn = (z_ref[...] - mu) * (inv * gam_ref[...]) + bet_ref[...]
    zl = jnp.where(zn >= 0, zn, 0.01 * zn)
    b2 = b2_ref[...]
    oA_ref[...] = (jnp.dot(zl, w2A_ref[...], preferred_element_type=jnp.float32)
                   + b2[:, :DH] + hA_ref[...])
    oB_ref[...] = (jnp.dot(zl, w2B_ref[...], preferred_element_type=jnp.float32)
                   + b2[:, DH:] + hB_ref[...])


def _layer_k2(z, ssum, ssq, gam, bet, W2, b2, hA, hB):
    R = 2000
    D2 = 2 * D
    return pl.pallas_call(
        _k2_body,
        grid=(N // R,),
        in_specs=[
            pl.BlockSpec((R, D2), lambda i: (i, 0)),
            pl.BlockSpec((1, D2), lambda i: (0, 0)),
            pl.BlockSpec((1, D2), lambda i: (0, 0)),
            pl.BlockSpec((1, D2), lambda i: (0, 0)),
            pl.BlockSpec((1, D2), lambda i: (0, 0)),
            pl.BlockSpec((D2, DH), lambda i: (0, 0)),
            pl.BlockSpec((D2, DH), lambda i: (0, 0)),
            pl.BlockSpec((1, D), lambda i: (0, 0)),
            pl.BlockSpec((R, DH), lambda i: (i, 0)),
            pl.BlockSpec((R, DH), lambda i: (i, 0)),
        ],
        out_specs=[
            pl.BlockSpec((R, DH), lambda i: (i, 0)),
            pl.BlockSpec((R, DH), lambda i: (i, 0)),
        ],
        out_shape=[
            jax.ShapeDtypeStruct((N, DH), jnp.float32),
            jax.ShapeDtypeStruct((N, DH), jnp.float32),
        ],
    )(z, ssum, ssq, gam.reshape(1, D2), bet.reshape(1, D2),
      W2[:, :DH], W2[:, DH:], b2.reshape(1, D), hA, hB)


def _pool_body(hA_ref, hB_ref, b_ref, wgA_ref, wgB_ref, bg_ref,
               wfA_ref, wfB_ref, bf_ref, o_ref):
    hA = hA_ref[...]
    hB = hB_ref[...]
    gate = (jnp.sum(hA * wgA_ref[...], axis=1, keepdims=True)
            + jnp.sum(hB * wgB_ref[...], axis=1, keepdims=True)
            + bg_ref[0, 0])
    seg = b_ref[...]
    gids = lax.broadcasted_iota(jnp.int32, (N, G), 1)
    M = seg == gids
    gm = jnp.max(jnp.where(M, gate, jnp.float32(-1e30)), axis=0, keepdims=True)
    gmax_n = jnp.sum(jnp.where(M, gm, 0.0), axis=1, keepdims=True)
    ex = jnp.exp(gate - gmax_n)
    den = jnp.sum(jnp.where(M, ex, 0.0), axis=0, keepdims=True)
    den_n = jnp.sum(jnp.where(M, den, 0.0), axis=1, keepdims=True)
    w = ex / den_n
    Mf = M.astype(jnp.float32)
    dn = (((0,), (0,)), ((), ()))
    pA = lax.dot_general(Mf, hA * w, dn, preferred_element_type=jnp.float32)
    pB = lax.dot_general(Mf, hB * w, dn, preferred_element_type=jnp.float32)
    t = (jnp.dot(pA, wfA_ref[...], preferred_element_type=jnp.float32)
         + jnp.dot(pB, wfB_ref[...], preferred_element_type=jnp.float32)
         + bf_ref[0, 0])
    o_ref[...] = 1.0 / (1.0 + jnp.exp(-t))


def _pool(hA, hB, batch32, Wg, bg, Wf, bf):
    return pl.pallas_call(
        _pool_body,
        out_shape=jax.ShapeDtypeStruct((G, 1), jnp.float32),
    )(hA, hB, batch32, Wg[:DH, 0].reshape(1, DH), Wg[DH:, 0].reshape(1, DH),
      bg.reshape(1, 1),
      Wf[:DH, :], Wf[DH:, :], bf.reshape(1, 1))


def kernel(x, edge_index, edge_attr, batch, We, be, Wee, bee, Wc, bc,
           W1, b1, gam, bet, W2, b2, Wg, bg, Wf, bf):
    src = edge_index[0].astype(jnp.int32)
    dst = edge_index[1].astype(jnp.int32)
    sdx = jnp.stack([src.reshape(E // K, K), dst.reshape(E // K, K)], axis=1)
    batch32 = batch.astype(jnp.int32).reshape(N, 1)
    zrows = jnp.zeros((N, DH), jnp.float32)

    hA, hB = _node_encode(x, We, be)
    eA, eB = _edge_encode(edge_attr, Wee, bee)

    for i in range(Wc.shape[0]):
        aA, aB = _edge_phase(hA, hB, eA, eB, sdx, zrows)
        z, ssum, ssq = _layer_k1(hA, hB, aA, aB, Wc[i], bc[i], W1[i], b1[i])
        hA, hB = _layer_k2(z, ssum, ssq, gam[i], bet[i], W2[i], b2[i], hA, hB)

    return _pool(hA, hB, batch32, Wg, bg, Wf, bf)
